# baseline (device time: 496948 ns/iter reference)
import jax
import jax.numpy as jnp
from jax import lax
from jax.experimental import pallas as pl
from jax.experimental.pallas import tpu as pltpu

HK = 16
LK = 2 * HK
S = 3
S2 = 2
SB = 6
XA = 8
YA = 8
LEAD = 3


def kernel(x):
    m, n = x.shape
    h = m // 2
    r = h // HK

    def body(x_ref, out_ref, stage, arena, rot, xarena, yarena,
             in_sems, out_sems, xplace_sems, yplace_sems,
             x_send_sems, x_recv_sems, y_send_sems, y_recv_sems,
             x_credit, y_credit):
        my_x = lax.axis_index("x")
        my_y = lax.axis_index("y")
        other_x = 1 - my_x
        other_y = 1 - my_y
        base_mine = my_x * m
        base_rem = other_x * m

        def chunk_rows(c):
            half = my_y if c < HK else other_y
            return half * h + (c % HK) * r

        barrier_sem = pltpu.get_barrier_semaphore()
        for nbr in [(other_x, my_y), (my_x, other_y)]:
            pl.semaphore_signal(
                barrier_sem, inc=1,
                device_id=nbr, device_id_type=pl.DeviceIdType.MESH,
            )
        pl.semaphore_wait(barrier_sem, 2)

        def stage_in(c):
            cp = pltpu.make_async_copy(
                x_ref.at[pl.ds(chunk_rows(c), r)],
                stage.at[c % S],
                in_sems.at[c % S],
            )
            cp.start()
            return cp

        ins = [None] * LK
        outs = [None] * LK
        x_sends = []
        y_sends = []
        x_places = []
        y_places = []

        def recv_step(c):
            rows = pl.ds(base_rem + my_y * h + c * r, r)
            recv = pltpu.make_async_remote_copy(
                src_ref=xarena.at[c % XA],
                dst_ref=xarena.at[c % XA],
                send_sem=x_send_sems.at[c],
                recv_sem=x_recv_sems.at[c],
                device_id=(other_x, my_y),
                device_id_type=pl.DeviceIdType.MESH,
            )
            recv.wait_recv()
            if c >= YA:
                pl.semaphore_wait(y_credit, 1)
            f = pltpu.make_async_remote_copy(
                src_ref=xarena.at[c % XA],
                dst_ref=yarena.at[c % YA],
                send_sem=y_send_sems.at[c],
                recv_sem=y_recv_sems.at[c],
                device_id=(my_x, other_y),
                device_id_type=pl.DeviceIdType.MESH,
            )
            f.start()
            y_sends.append(f)
            lc = pltpu.make_async_copy(
                xarena.at[c % XA], out_ref.at[rows], xplace_sems.at[c]
            )
            lc.start()
            x_places.append(lc)

        def x_free_step(c):
            y_sends[c].wait_send()
            x_places[c].wait()
            if c < HK - XA:
                pl.semaphore_signal(
                    x_credit, inc=1,
                    device_id=(other_x, my_y),
                    device_id_type=pl.DeviceIdType.MESH,
                )

        def y_drain_step(c):
            recv = pltpu.make_async_remote_copy(
                src_ref=yarena.at[c % YA],
                dst_ref=yarena.at[c % YA],
                send_sem=y_send_sems.at[c],
                recv_sem=y_recv_sems.at[c],
                device_id=(my_x, other_y),
                device_id_type=pl.DeviceIdType.MESH,
            )
            recv.wait_recv()
            lc = pltpu.make_async_copy(
                yarena.at[c % YA],
                out_ref.at[pl.ds(base_rem + other_y * h + c * r, r)],
                yplace_sems.at[c],
            )
            lc.start()
            y_places.append(lc)

        def y_free_step(c):
            y_places[c].wait()
            if c < HK - YA:
                pl.semaphore_signal(
                    y_credit, inc=1,
                    device_id=(my_x, other_y),
                    device_id_type=pl.DeviceIdType.MESH,
                )

        for c in range(S):
            ins[c] = stage_in(c)

        for c in range(LK):
            ins[c].wait()
            if c < HK:
                if c >= SB:
                    x_sends[c - SB].wait_send()
                    outs[c - SB].wait()
                arena[c % SB] = stage[c % S].astype(jnp.bfloat16)
                src = arena.at[c % SB]
            else:
                j = c - HK
                if j >= S2:
                    outs[HK + j - S2].wait()
                rot[j % S2] = stage[c % S].astype(jnp.bfloat16)
                src = rot.at[j % S2]
            nxt = c + S
            if nxt < LK:
                ins[nxt] = stage_in(nxt)
            outs[c] = pltpu.make_async_copy(
                src, out_ref.at[pl.ds(base_mine + chunk_rows(c), r)],
                out_sems.at[c],
            )
            outs[c].start()
            if c < HK:
                if c >= XA:
                    pl.semaphore_wait(x_credit, 1)
                s = pltpu.make_async_remote_copy(
                    src_ref=arena.at[c % SB],
                    dst_ref=xarena.at[c % XA],
                    send_sem=x_send_sems.at[c],
                    recv_sem=x_recv_sems.at[c],
                    device_id=(other_x, my_y),
                    device_id_type=pl.DeviceIdType.MESH,
                )
                s.start()
                x_sends.append(s)
            cr = c - LEAD
            if 0 <= cr < HK:
                recv_step(cr)
            cy = c - LEAD - 1
            if 0 <= cy < HK:
                y_drain_step(cy)
            cf = c - LEAD - 2
            if 0 <= cf < HK:
                x_free_step(cf)
            cyf = c - LEAD - 3
            if 0 <= cyf < HK:
                y_free_step(cyf)

        for c in range(max(0, HK - SB), HK):
            x_sends[c].wait_send()
            outs[c].wait()
        for c in range(LK - S2, LK):
            outs[c].wait()

    return pl.pallas_call(
        body,
        out_shape=jax.ShapeDtypeStruct((2 * m, n), jnp.bfloat16),
        in_specs=[pl.BlockSpec(memory_space=pl.ANY)],
        out_specs=pl.BlockSpec(memory_space=pl.ANY),
        scratch_shapes=[
            pltpu.VMEM((S, h // HK, n), jnp.float32),
            pltpu.VMEM((SB, h // HK, n), jnp.bfloat16),
            pltpu.VMEM((S2, h // HK, n), jnp.bfloat16),
            pltpu.VMEM((XA, h // HK, n), jnp.bfloat16),
            pltpu.VMEM((YA, h // HK, n), jnp.bfloat16),
            pltpu.SemaphoreType.DMA((S,)),
            pltpu.SemaphoreType.DMA((LK,)),
            pltpu.SemaphoreType.DMA((HK,)),
            pltpu.SemaphoreType.DMA((HK,)),
            pltpu.SemaphoreType.DMA((HK,)),
            pltpu.SemaphoreType.DMA((HK,)),
            pltpu.SemaphoreType.DMA((HK,)),
            pltpu.SemaphoreType.DMA((HK,)),
            pltpu.SemaphoreType.REGULAR,
            pltpu.SemaphoreType.REGULAR,
        ],
        compiler_params=pltpu.CompilerParams(
            collective_id=0, vmem_limit_bytes=100 * 1024 * 1024
        ),
    )(x)


# device time: 480266 ns/iter; 1.0347x vs baseline; 1.0347x over previous
import jax
import jax.numpy as jnp
from jax import lax
from jax.experimental import pallas as pl
from jax.experimental.pallas import tpu as pltpu

HK = 32
LK = 2 * HK
S = 3
S2 = 2
SB = 6
LEAD = 3


def kernel(x):
    m, n = x.shape
    h = m // 2
    r = h // HK

    def body(x_ref, out_ref, stage, arena, rot, xarena, in_sems, out_sems,
             out2_sems, x_send_sems, x_recv_sems, y_send_sems, y_recv_sems):
        my_x = lax.axis_index("x")
        my_y = lax.axis_index("y")
        other_x = 1 - my_x
        other_y = 1 - my_y
        base_mine = my_x * m
        base_rem = other_x * m

        def chunk_rows(c):
            half = my_y if c < HK else other_y
            return half * h + (c % HK) * r

        barrier_sem = pltpu.get_barrier_semaphore()
        for nbr in [(other_x, my_y), (my_x, other_y)]:
            pl.semaphore_signal(
                barrier_sem, inc=1,
                device_id=nbr, device_id_type=pl.DeviceIdType.MESH,
            )
        pl.semaphore_wait(barrier_sem, 2)

        def stage_in(c):
            cp = pltpu.make_async_copy(
                x_ref.at[pl.ds(chunk_rows(c), r)],
                stage.at[c % S],
                in_sems.at[c % S],
            )
            cp.start()
            return cp

        def recv_step(c):
            rows = pl.ds(base_rem + my_y * h + c * r, r)
            recv = pltpu.make_async_remote_copy(
                src_ref=xarena.at[c],
                dst_ref=xarena.at[c],
                send_sem=x_send_sems.at[c],
                recv_sem=x_recv_sems.at[c],
                device_id=(other_x, my_y),
                device_id_type=pl.DeviceIdType.MESH,
            )
            recv.wait_recv()
            f = pltpu.make_async_remote_copy(
                src_ref=xarena.at[c],
                dst_ref=out_ref.at[rows],
                send_sem=y_send_sems.at[c],
                recv_sem=y_recv_sems.at[c],
                device_id=(my_x, other_y),
                device_id_type=pl.DeviceIdType.MESH,
            )
            f.start()
            lc = pltpu.make_async_copy(
                xarena.at[c], out_ref.at[rows], out2_sems.at[c]
            )
            lc.start()
            return f, lc

        ins = [None] * LK
        outs = [None] * LK
        x_sends = []
        y_sends = []
        x_places = []
        for c in range(S):
            ins[c] = stage_in(c)

        for c in range(LK):
            ins[c].wait()
            if c < HK:
                if c >= SB:
                    x_sends[c - SB].wait_send()
                    outs[c - SB].wait()
                arena[c % SB] = stage[c % S].astype(jnp.bfloat16)
                src = arena.at[c % SB]
            else:
                j = c - HK
                if j >= S2:
                    outs[HK + j - S2].wait()
                rot[j % S2] = stage[c % S].astype(jnp.bfloat16)
                src = rot.at[j % S2]
            nxt = c + S
            if nxt < LK:
                ins[nxt] = stage_in(nxt)
            outs[c] = pltpu.make_async_copy(
                src, out_ref.at[pl.ds(base_mine + chunk_rows(c), r)],
                out_sems.at[c],
            )
            outs[c].start()
            if c < HK:
                s = pltpu.make_async_remote_copy(
                    src_ref=arena.at[c % SB],
                    dst_ref=xarena.at[c],
                    send_sem=x_send_sems.at[c],
                    recv_sem=x_recv_sems.at[c],
                    device_id=(other_x, my_y),
                    device_id_type=pl.DeviceIdType.MESH,
                )
                s.start()
                x_sends.append(s)
            if LEAD <= c < HK + LEAD:
                f, lc = recv_step(c - LEAD)
                y_sends.append(f)
                x_places.append(lc)

        for c in range(HK):
            recv = pltpu.make_async_remote_copy(
                src_ref=xarena.at[c],
                dst_ref=out_ref.at[pl.ds(base_rem + other_y * h + c * r, r)],
                send_sem=y_send_sems.at[c],
                recv_sem=y_recv_sems.at[c],
                device_id=(my_x, other_y),
                device_id_type=pl.DeviceIdType.MESH,
            )
            recv.wait_recv()

        for c in range(LK - S2, LK):
            outs[c].wait()
        for c in range(HK - SB, HK):
            outs[c].wait()
        for lc in x_places:
            lc.wait()
        for s in x_sends[HK - SB:]:
            s.wait_send()
        for s in y_sends:
            s.wait_send()

    return pl.pallas_call(
        body,
        out_shape=jax.ShapeDtypeStruct((2 * m, n), jnp.bfloat16),
        in_specs=[pl.BlockSpec(memory_space=pl.ANY)],
        out_specs=pl.BlockSpec(memory_space=pl.ANY),
        scratch_shapes=[
            pltpu.VMEM((S, h // HK, n), jnp.float32),
            pltpu.VMEM((SB, h // HK, n), jnp.bfloat16),
            pltpu.VMEM((S2, h // HK, n), jnp.bfloat16),
            pltpu.VMEM((HK, h // HK, n), jnp.bfloat16),
            pltpu.SemaphoreType.DMA((S,)),
            pltpu.SemaphoreType.DMA((LK,)),
            pltpu.SemaphoreType.DMA((HK,)),
            pltpu.SemaphoreType.DMA((HK,)),
            pltpu.SemaphoreType.DMA((HK,)),
            pltpu.SemaphoreType.DMA((HK,)),
            pltpu.SemaphoreType.DMA((HK,)),
        ],
        compiler_params=pltpu.CompilerParams(
            collective_id=0, vmem_limit_bytes=100 * 1024 * 1024
        ),
    )(x)
